# baseline (speedup 1.0000x reference)
import functools

import jax
import jax.numpy as jnp
from jax import lax
from jax.experimental import pallas as pl
from jax.experimental.pallas import tpu as pltpu
from jax.experimental.pallas import tpu_sc as plsc

_N = 10000
_K = 16

_BIG_F = 3.0e38
_BIG_I = 2**30


def _knn_kernel_body(nc, xr_ref, xc_ref, idx_ref, rv_ref, ri_ref, cv_ref,
                     ci_ref, *, bm, bn, n_valid):
    j = pl.program_id(1)
    i = pl.program_id(0)

    @pl.when(j == 0)
    def _init():
        rv_ref[...] = jnp.full((_K, bm), _BIG_F, jnp.float32)
        ri_ref[...] = jnp.zeros((_K, bm), jnp.int32)

    xr = xr_ref[...]
    xc = xc_ref[...]
    d2c = jnp.sum(xc * xc, axis=1, keepdims=True)
    d2r = jnp.sum(xr * xr, axis=1)[None, :]
    dots = lax.dot_general(
        xc, xr, (((1,), (1,)), ((), ())), preferred_element_type=jnp.float32
    )
    dist = (d2r + d2c) - 2.0 * dots
    col_ids = j * bn + lax.broadcasted_iota(jnp.int32, (bn, bm), 0)
    row_ids = i * bm + lax.broadcasted_iota(jnp.int32, (bn, bm), 1)
    invalid = (col_ids == row_ids) | (col_ids >= n_valid)
    cv_ref[:_K, :] = rv_ref[...]
    cv_ref[_K:, :] = jnp.where(invalid, _BIG_F, dist)
    ci_ref[:_K, :] = ri_ref[...]
    ci_ref[_K:, :] = col_ids

    ns = bn + _K
    chunks = []
    s = 0
    while s < ns:
        chunks.append((s, min(64, ns - s)))
        s += min(64, ns - s)
    siota = lax.broadcasted_iota(jnp.int32, (_K, bm), 0)

    def _extract(t, carry):
        nv, ni = carry
        m = jnp.full((1, bm), _BIG_F, jnp.float32)
        for (s, sz) in chunks:
            m = jnp.minimum(
                m, jnp.min(cv_ref[pl.ds(s, sz), :], axis=0, keepdims=True))
        sel = jnp.full((1, bm), _BIG_I, jnp.int32)
        for (s, sz) in chunks:
            cvc = cv_ref[pl.ds(s, sz), :]
            cic = ci_ref[pl.ds(s, sz), :]
            sel = jnp.minimum(
                sel,
                jnp.min(jnp.where(cvc == m, cic, _BIG_I), axis=0,
                        keepdims=True))
        for (s, sz) in chunks:
            cvc = cv_ref[pl.ds(s, sz), :]
            cic = ci_ref[pl.ds(s, sz), :]
            cv_ref[pl.ds(s, sz), :] = jnp.where(
                (cvc == m) & (cic == sel), _BIG_F, cvc)
        nv = jnp.where(siota == t, m, nv)
        ni = jnp.where(siota == t, sel, ni)
        return nv, ni

    nv0 = jnp.full((_K, bm), _BIG_F, jnp.float32)
    ni0 = jnp.zeros((_K, bm), jnp.int32)
    nv, ni = lax.fori_loop(0, _K, _extract, (nv0, ni0))
    rv_ref[...] = nv
    ri_ref[...] = ni

    @pl.when(j == nc - 1)
    def _write():
        idx_ref[...] = ni


@functools.partial(jax.jit, static_argnames=("interpret",))
def _knn_idx_pallas(x, interpret=False):
    n, d = x.shape
    bm = 128
    bn = 512
    npad = ((n + bn - 1) // bn) * bn
    xp = jnp.pad(x, ((0, npad - n), (0, 0)))
    nr = npad // bm
    nc = npad // bn
    body = functools.partial(_knn_kernel_body, nc, bm=bm, bn=bn, n_valid=n)
    idx = pl.pallas_call(
        body,
        grid=(nr, nc),
        in_specs=[
            pl.BlockSpec((bm, d), lambda i, j: (i, 0)),
            pl.BlockSpec((bn, d), lambda i, j: (j, 0)),
        ],
        out_specs=pl.BlockSpec((_K, bm), lambda i, j: (0, i)),
        out_shape=jax.ShapeDtypeStruct((_K, npad), jnp.int32),
        scratch_shapes=[
            pltpu.VMEM((_K, bm), jnp.float32),
            pltpu.VMEM((_K, bm), jnp.int32),
            pltpu.VMEM((bn + _K, bm), jnp.float32),
            pltpu.VMEM((bn + _K, bm), jnp.int32),
        ],
        interpret=interpret,
    )(xp, xp)
    return idx[:, :n]


@functools.partial(jax.jit, static_argnames=("cs",))
def _sc_gather(table, idx_flat, cs=40):
    d_orig = table.shape[1]
    dpad = ((d_orig + 127) // 128) * 128
    if dpad != d_orig:
        table = jnp.pad(table, ((0, 0), (0, dpad - d_orig)))
    b_total = idx_flat.shape[0]
    d = table.shape[1]
    info = plsc.get_sparse_core_info()
    nw = info.num_cores * info.num_subcores
    b_per_w = b_total // nw
    n_chunks = b_per_w // cs
    mesh = plsc.VectorSubcoreMesh(core_axis_name="c", subcore_axis_name="s")

    @functools.partial(
        pl.kernel,
        mesh=mesh,
        out_type=jax.ShapeDtypeStruct((b_total, d), jnp.float32),
        scratch_types=[
            pltpu.VMEM((cs,), jnp.int32),
            pltpu.VMEM((cs, d), jnp.float32),
            pltpu.SemaphoreType.DMA,
        ],
    )
    def _gather_kernel(table_hbm, idx_hbm, out_hbm, idx_v, rows_v, sem):
        wid = lax.axis_index("s") * info.num_cores + lax.axis_index("c")
        base = wid * b_per_w

        def _chunk(g, carry):
            off = base + g * cs
            pltpu.sync_copy(idx_hbm.at[pl.ds(off, cs)], idx_v)
            pltpu.async_copy(table_hbm.at[idx_v], rows_v, sem).wait()
            pltpu.sync_copy(rows_v, out_hbm.at[pl.ds(off, cs)])
            return carry

        lax.fori_loop(0, n_chunks, _chunk, 0)

    out = _gather_kernel(table, idx_flat)
    return out[:, :d_orig] if dpad != d_orig else out


def _mm_body(x_ref, w_ref, o_ref):
    o_ref[...] = lax.dot_general(
        x_ref[...], w_ref[...], (((1,), (0,)), ((), ())),
        preferred_element_type=jnp.float32)


@jax.jit
def _matmul_pallas(x, w):
    n, din = x.shape
    dout = w.shape[1]
    bm = 400
    return pl.pallas_call(
        _mm_body,
        grid=(n // bm,),
        in_specs=[
            pl.BlockSpec((bm, din), lambda i: (i, 0)),
            pl.BlockSpec((din, dout), lambda i: (0, 0)),
        ],
        out_specs=pl.BlockSpec((bm, dout), lambda i: (i, 0)),
        out_shape=jax.ShapeDtypeStruct((n, dout), jnp.float32),
    )(x, w)


def _attn_body(x_ref, wl_ref, hrn_ref, a_ref, b_ref, o_ref, hl_ref, *,
               bm, dout):
    hl_ref[...] = lax.dot_general(
        x_ref[...], wl_ref[...], (((1,), (0,)), ((), ())),
        preferred_element_type=jnp.float32)
    a = a_ref[...]
    es = []
    for kk in range(_K):
        mm = hl_ref[...] + hrn_ref[kk]
        mm = jnp.where(mm >= 0.0, mm, 0.2 * mm)
        es.append(lax.dot_general(mm, a, (((1,), (0,)), ((), ())),
                                  preferred_element_type=jnp.float32))
    emax = es[0]
    for kk in range(1, _K):
        emax = jnp.maximum(emax, es[kk])
    ws = [jnp.exp(e - emax) for e in es]
    dn = list(ws)
    while len(dn) > 1:
        h = len(dn) // 2
        dn = [dn[i] + dn[i + h] for i in range(h)]
    denom = dn[0]
    alphas = [w / denom for w in ws]
    dc = 128
    for c0 in range(0, dout, dc):
        sz = min(dc, dout - c0)
        ts = [alphas[kk] * hrn_ref[kk, :, pl.ds(c0, sz)] for kk in range(_K)]
        while len(ts) > 1:
            h = len(ts) // 2
            ts = [ts[i] + ts[i + h] for i in range(h)]
        o_ref[:, pl.ds(c0, sz)] = ts[0] + b_ref[:, pl.ds(c0, sz)]


@jax.jit
def _attn_pallas(x, wl, hrn, a, b):
    n, din = x.shape
    dout = wl.shape[1]
    bm = 80
    body = functools.partial(_attn_body, bm=bm, dout=dout)
    return pl.pallas_call(
        body,
        grid=(n // bm,),
        in_specs=[
            pl.BlockSpec((bm, din), lambda i: (i, 0)),
            pl.BlockSpec((din, dout), lambda i: (0, 0)),
            pl.BlockSpec((_K, bm, dout), lambda i: (0, i, 0)),
            pl.BlockSpec((dout, 1), lambda i: (0, 0)),
            pl.BlockSpec((1, dout), lambda i: (0, 0)),
        ],
        out_specs=pl.BlockSpec((bm, dout), lambda i: (i, 0)),
        out_shape=jax.ShapeDtypeStruct((n, dout), jnp.float32),
        scratch_shapes=[pltpu.VMEM((bm, dout), jnp.float32)],
    )(x, wl, hrn, a.reshape(dout, 1), b.reshape(1, dout))


def _mlp_body(x_ref, h1_ref, h2_ref, h3_ref, h4_ref, w1_ref, b1_ref, w2_ref,
              b2_ref, w3_ref, b3_ref, o_ref, *, dims):
    d0, d1, d2, d3, d4 = dims
    o1 = d0
    y = lax.dot_general(x_ref[...], w1_ref[pl.ds(0, d0), :],
                        (((1,), (0,)), ((), ())),
                        preferred_element_type=jnp.float32)
    for h_ref, dd in ((h1_ref, d1), (h2_ref, d2), (h3_ref, d3), (h4_ref, d4)):
        y = y + lax.dot_general(h_ref[...], w1_ref[pl.ds(o1, dd), :],
                                (((1,), (0,)), ((), ())),
                                preferred_element_type=jnp.float32)
        o1 += dd
    y = jnp.maximum(y + b1_ref[...], 0.0)
    y = lax.dot_general(y, w2_ref[...], (((1,), (0,)), ((), ())),
                        preferred_element_type=jnp.float32)
    y = jnp.maximum(y + b2_ref[...], 0.0)
    y = lax.dot_general(y, w3_ref[...], (((1,), (0,)), ((), ())),
                        preferred_element_type=jnp.float32)
    o_ref[...] = y + b3_ref[...]


@jax.jit
def _mlp_pallas(x, h1, h2, h3, h4, w1, b1, w2, b2, w3, b3):
    n = x.shape[0]
    dims = (x.shape[1], h1.shape[1], h2.shape[1], h3.shape[1], h4.shape[1])
    cat = sum(dims)
    dh1 = w1.shape[1]
    dh2 = w2.shape[1]
    dh3 = w3.shape[1]
    bm = 400
    body = functools.partial(_mlp_body, dims=dims)
    return pl.pallas_call(
        body,
        grid=(n // bm,),
        in_specs=[
            pl.BlockSpec((bm, dims[0]), lambda i: (i, 0)),
            pl.BlockSpec((bm, dims[1]), lambda i: (i, 0)),
            pl.BlockSpec((bm, dims[2]), lambda i: (i, 0)),
            pl.BlockSpec((bm, dims[3]), lambda i: (i, 0)),
            pl.BlockSpec((bm, dims[4]), lambda i: (i, 0)),
            pl.BlockSpec((cat, dh1), lambda i: (0, 0)),
            pl.BlockSpec((1, dh1), lambda i: (0, 0)),
            pl.BlockSpec((dh1, dh2), lambda i: (0, 0)),
            pl.BlockSpec((1, dh2), lambda i: (0, 0)),
            pl.BlockSpec((dh2, dh3), lambda i: (0, 0)),
            pl.BlockSpec((1, dh3), lambda i: (0, 0)),
        ],
        out_specs=pl.BlockSpec((bm, dh3), lambda i: (i, 0)),
        out_shape=jax.ShapeDtypeStruct((n, dh3), jnp.float32),
    )(x, h1, h2, h3, h4, w1, b1.reshape(1, dh1), w2, b2.reshape(1, dh2),
      w3, b3.reshape(1, dh3))


def _gatv2_layer_exact(x, Wl, Wr, a, b):
    n = x.shape[0]
    dout = Wr.shape[1]
    idx_t = _knn_idx_pallas(x)
    hl = x @ Wl
    hr = x @ Wr
    idx_nm = idx_t.T.reshape(-1)
    hrn = _sc_gather(hr, idx_nm).reshape(n, _K, dout)
    m = jax.nn.leaky_relu(hl[:, None, :] + hrn, negative_slope=0.2)
    e = jnp.einsum('nkd,d->nk', m, a)
    alpha = jax.nn.softmax(e, axis=1)
    return jnp.sum(alpha[:, :, None] * hrn, axis=1) + b


def _gatv2_layer_fused(x, Wl, Wr, a, b):
    n = x.shape[0]
    dout = Wr.shape[1]
    idx_t = _knn_idx_pallas(x)
    hr = _matmul_pallas(x, Wr)
    hrn = _sc_gather(hr, idx_t.reshape(-1)).reshape(_K, n, dout)
    return _attn_pallas(x, Wl, hrn, a, b)


def kernel(x, Wl1, Wr1, a1, b1, Wl2, Wr2, a2, b2, Wl3, Wr3, a3, b3,
           Wl4, Wr4, a4, b4, Wm1, bm1, Wm2, bm2, Wm3, bm3):
    outs = [x]
    params = [(Wl1, Wr1, a1, b1), (Wl2, Wr2, a2, b2), (Wl3, Wr3, a3, b3)]
    for (Wl, Wr, a, b) in params:
        outs.append(_gatv2_layer_exact(outs[-1], Wl, Wr, a, b))
    outs.append(_gatv2_layer_fused(outs[-1], Wl4, Wr4, a4, b4))
    return _mlp_pallas(outs[0], outs[1], outs[2], outs[3], outs[4],
                       Wm1, bm1, Wm2, bm2, Wm3, bm3)
